# baseline (speedup 1.0000x reference)
import functools

import jax
import jax.numpy as jnp
from jax import lax
from jax.experimental import pallas as pl
from jax.experimental.pallas import tpu as pltpu
from jax.experimental.pallas import tpu_sc as plsc

B, S, DIM = 4, 8192, 128
N = B * S

_info = plsc.get_sparse_core_info()
NC, NS, L = _info.num_cores, _info.num_subcores, _info.num_lanes
NW = NC * NS
C = N // NW
H = C // 2
BLK = 64
BLK_SHIFT = 6
NB = H // BLK
CAP = H + BLK
ROWS = H + 4 * BLK

_mesh = plsc.VectorSubcoreMesh(core_axis_name="c", subcore_axis_name="s")


@functools.partial(
    pl.kernel,
    mesh=_mesh,
    compiler_params=pltpu.CompilerParams(needs_layout_passes=False),
    out_type=jax.ShapeDtypeStruct((N, DIM), jnp.float32),
    scratch_types=(
        [pltpu.VMEM((C,), jnp.int32), pltpu.VMEM((C,), jnp.int32)]
        + [pltpu.VMEM((CAP,), jnp.int32) for _ in range(4)]
        + [pltpu.VMEM((CAP,), jnp.int32) for _ in range(4)]
        + [pltpu.VMEM((NB, BLK), jnp.int32) for _ in range(4)]
        + [pltpu.VMEM((ROWS, DIM), jnp.float32)]
        + [pltpu.SemaphoreType.DMA, pltpu.SemaphoreType.DMA]
    ),
)
def _sc_lookup(ids_hbm, mods_hbm, t0, t1, t2, t3, out_hbm, *scratch):
    ids_v, mods_v = scratch[0], scratch[1]
    idl = scratch[2:6]
    posl = scratch[6:10]
    pos2d = scratch[10:14]
    rows = scratch[14]
    gsem, ssem = scratch[15], scratch[16]
    tables = (t0, t1, t2, t3)

    wid = lax.axis_index("s") * NC + lax.axis_index("c")
    base = wid * C
    pltpu.sync_copy(ids_hbm.at[pl.ds(base, C)], ids_v)
    pltpu.sync_copy(mods_hbm.at[pl.ds(base, C)], mods_v)

    pending = []

    for h in range(2):
        offs = [jnp.zeros((L,), jnp.int32) for _ in range(4)]
        for i in range(H // L):
            sl = pl.ds(h * H + i * L, L)
            iv = ids_v[sl]
            mv = mods_v[sl]
            pv = jnp.arange(L, dtype=jnp.int32) + (base + h * H + i * L)
            for t in range(4):
                m = mv == t
                dst = plsc.cumsum(m.astype(jnp.int32)) + (offs[t] - 1)
                plsc.store_scatter(idl[t], [dst], iv, mask=m)
                plsc.store_scatter(posl[t], [dst], pv, mask=m)
                offs[t] = offs[t] + plsc.all_reduce_population_count(m)

        cnt = [jnp.max(offs[t]) for t in range(4)]

        for t in range(4):

            @pl.when(cnt[t] > 0)
            def _(t=t):
                last = jnp.broadcast_to(cnt[t] - 1, (L,)).astype(jnp.int32)
                idsp = plsc.load_gather(idl[t], [last])
                possp = plsc.load_gather(posl[t], [last])
                for g in range(BLK // L):
                    fill = cnt[t] + jnp.arange(L, dtype=jnp.int32) + g * L
                    plsc.store_scatter(idl[t], [fill], idsp)
                    plsc.store_scatter(posl[t], [fill], possp)

        nbk = [
            lax.shift_right_logical(cnt[t] + (BLK - 1), BLK_SHIFT) for t in range(4)
        ]
        seg = [None] * 4
        seg[0] = jnp.int32(0)
        for t in range(1, 4):
            seg[t] = seg[t - 1] + lax.shift_left(nbk[t - 1], BLK_SHIFT)

        for cond, dsl, tp, bp in pending:

            @pl.when(cond)
            def _(dsl=dsl, tp=tp, bp=bp):
                pltpu.make_async_copy(
                    rows.at[dsl], out_hbm.at[pos2d[tp].at[bp]], ssem
                ).wait()
        pending = []

        for t in range(4):
            for b in range(NB):
                for g in range(BLK // L):
                    pos2d[t][b, pl.ds(g * L, L)] = posl[t][pl.ds(b * BLK + g * L, L)]

        for t in range(4):
            for b in range(NB):

                @pl.when(b < nbk[t])
                def _(t=t, b=b):
                    pltpu.async_copy(
                        tables[t].at[idl[t].at[pl.ds(b * BLK, BLK)]],
                        rows.at[pl.ds(seg[t] + b * BLK, BLK)],
                        gsem,
                    )
        for t in range(4):
            for b in range(NB):

                @pl.when(b < nbk[t])
                def _(t=t, b=b):
                    pltpu.make_async_copy(
                        tables[t].at[idl[t].at[pl.ds(b * BLK, BLK)]],
                        rows.at[pl.ds(seg[t] + b * BLK, BLK)],
                        gsem,
                    ).wait()

        for t in range(4):
            for b in range(NB):
                cond = b < nbk[t]
                dsl = pl.ds(seg[t] + b * BLK, BLK)

                @pl.when(cond)
                def _(t=t, b=b, dsl=dsl):
                    pltpu.async_copy(rows.at[dsl], out_hbm.at[pos2d[t].at[b]], ssem)

                pending.append((cond, dsl, t, b))

    for cond, dsl, tp, bp in pending:

        @pl.when(cond)
        def _(dsl=dsl, tp=tp, bp=bp):
            pltpu.make_async_copy(
                rows.at[dsl], out_hbm.at[pos2d[tp].at[bp]], ssem
            ).wait()


def kernel(input_ids, modality_ids, text_table, image_table, video_table, audio_table):
    ids = input_ids.reshape(-1)
    mods = modality_ids.reshape(-1)
    out = _sc_lookup(ids, mods, text_table, image_table, video_table, audio_table)
    return out.reshape(B, S, DIM)


# speedup vs baseline: 1.2987x; 1.2987x over previous
import functools

import jax
import jax.numpy as jnp
from jax import lax
from jax.experimental import pallas as pl
from jax.experimental.pallas import tpu as pltpu
from jax.experimental.pallas import tpu_sc as plsc

B, S, DIM = 4, 8192, 128
N = B * S

_info = plsc.get_sparse_core_info()
NC, NS, L = _info.num_cores, _info.num_subcores, _info.num_lanes
NW = NC * NS
C = N // NW
H = C // 2
BLK = 32
BLK_SHIFT = 5
NB = H // BLK
CAP = H + BLK
ROWS = H + 4 * BLK

_mesh = plsc.VectorSubcoreMesh(core_axis_name="c", subcore_axis_name="s")


@functools.partial(
    pl.kernel,
    mesh=_mesh,
    compiler_params=pltpu.CompilerParams(needs_layout_passes=False),
    out_type=jax.ShapeDtypeStruct((N, DIM), jnp.float32),
    scratch_types=(
        [pltpu.VMEM((C,), jnp.int32), pltpu.VMEM((C,), jnp.int32)]
        + [pltpu.VMEM((CAP,), jnp.int32) for _ in range(4)]
        + [pltpu.VMEM((CAP,), jnp.int32) for _ in range(4)]
        + [pltpu.VMEM((NB, BLK), jnp.int32) for _ in range(4)]
        + [pltpu.VMEM((ROWS, DIM), jnp.float32)]
        + [pltpu.SemaphoreType.DMA, pltpu.SemaphoreType.DMA]
    ),
)
def _sc_lookup(ids_hbm, mods_hbm, t0, t1, t2, t3, out_hbm, *scratch):
    ids_v, mods_v = scratch[0], scratch[1]
    idl = scratch[2:6]
    posl = scratch[6:10]
    pos2d = scratch[10:14]
    rows = scratch[14]
    gsem, ssem = scratch[15], scratch[16]
    tables = (t0, t1, t2, t3)

    wid = lax.axis_index("s") * NC + lax.axis_index("c")
    base = wid * C
    pltpu.sync_copy(ids_hbm.at[pl.ds(base, C)], ids_v)
    pltpu.sync_copy(mods_hbm.at[pl.ds(base, C)], mods_v)

    pending = []

    for h in range(2):
        offs = [jnp.zeros((L,), jnp.int32) for _ in range(4)]
        for i in range(H // L):
            sl = pl.ds(h * H + i * L, L)
            iv = ids_v[sl]
            mv = mods_v[sl]
            pv = jnp.arange(L, dtype=jnp.int32) + (base + h * H + i * L)
            for t in range(4):
                m = mv == t
                dst = plsc.cumsum(m.astype(jnp.int32)) + (offs[t] - 1)
                plsc.store_scatter(idl[t], [dst], iv, mask=m)
                plsc.store_scatter(posl[t], [dst], pv, mask=m)
                offs[t] = offs[t] + plsc.all_reduce_population_count(m)

        cnt = [jnp.max(offs[t]) for t in range(4)]

        for t in range(4):

            @pl.when(cnt[t] > 0)
            def _(t=t):
                last = jnp.broadcast_to(cnt[t] - 1, (L,)).astype(jnp.int32)
                idsp = plsc.load_gather(idl[t], [last])
                possp = plsc.load_gather(posl[t], [last])
                for g in range(BLK // L):
                    fill = cnt[t] + jnp.arange(L, dtype=jnp.int32) + g * L
                    plsc.store_scatter(idl[t], [fill], idsp)
                    plsc.store_scatter(posl[t], [fill], possp)

        nbk = [
            lax.shift_right_logical(cnt[t] + (BLK - 1), BLK_SHIFT) for t in range(4)
        ]
        seg = [None] * 4
        seg[0] = jnp.int32(0)
        for t in range(1, 4):
            seg[t] = seg[t - 1] + lax.shift_left(nbk[t - 1], BLK_SHIFT)

        for cond, dsl, tp, bp in pending:

            @pl.when(cond)
            def _(dsl=dsl, tp=tp, bp=bp):
                pltpu.make_async_copy(
                    rows.at[dsl], out_hbm.at[pos2d[tp].at[bp]], ssem
                ).wait()
        pending = []

        for t in range(4):
            for b in range(NB):
                for g in range(BLK // L):
                    pos2d[t][b, pl.ds(g * L, L)] = posl[t][pl.ds(b * BLK + g * L, L)]

        for t in range(4):
            for b in range(NB):

                @pl.when(b < nbk[t])
                def _(t=t, b=b):
                    pltpu.async_copy(
                        tables[t].at[idl[t].at[pl.ds(b * BLK, BLK)]],
                        rows.at[pl.ds(seg[t] + b * BLK, BLK)],
                        gsem,
                    )
        for t in range(4):
            for b in range(NB):
                cond = b < nbk[t]
                dsl = pl.ds(seg[t] + b * BLK, BLK)

                @pl.when(cond)
                def _(t=t, b=b, dsl=dsl):
                    pltpu.make_async_copy(
                        tables[t].at[idl[t].at[pl.ds(b * BLK, BLK)]],
                        rows.at[dsl],
                        gsem,
                    ).wait()
                    pltpu.async_copy(rows.at[dsl], out_hbm.at[pos2d[t].at[b]], ssem)

                pending.append((cond, dsl, t, b))

    for cond, dsl, tp, bp in pending:

        @pl.when(cond)
        def _(dsl=dsl, tp=tp, bp=bp):
            pltpu.make_async_copy(
                rows.at[dsl], out_hbm.at[pos2d[tp].at[bp]], ssem
            ).wait()


def kernel(input_ids, modality_ids, text_table, image_table, video_table, audio_table):
    ids = input_ids.reshape(-1)
    mods = modality_ids.reshape(-1)
    out = _sc_lookup(ids, mods, text_table, image_table, video_table, audio_table)
    return out.reshape(B, S, DIM)


# speedup vs baseline: 1.6037x; 1.2349x over previous
import functools

import jax
import jax.numpy as jnp
from jax import lax
from jax.experimental import pallas as pl
from jax.experimental.pallas import tpu as pltpu
from jax.experimental.pallas import tpu_sc as plsc

B, S, DIM = 4, 8192, 128
N = B * S

_info = plsc.get_sparse_core_info()
NC, NS, L = _info.num_cores, _info.num_subcores, _info.num_lanes
NW = NC * NS
C = N // NW
H = C // 2
BLK = 32
BLK_SHIFT = 5
NB = H // BLK
CAP = H + BLK
ROWS = H + 4 * BLK

_mesh = plsc.VectorSubcoreMesh(core_axis_name="c", subcore_axis_name="s")


@functools.partial(
    pl.kernel,
    mesh=_mesh,
    compiler_params=pltpu.CompilerParams(needs_layout_passes=False),
    out_type=jax.ShapeDtypeStruct((N, DIM), jnp.float32),
    scratch_types=(
        [pltpu.VMEM((C,), jnp.int32), pltpu.VMEM((C,), jnp.int32)]
        + [pltpu.VMEM((CAP,), jnp.int32) for _ in range(4)]
        + [pltpu.VMEM((CAP,), jnp.int32) for _ in range(4)]
        + [pltpu.VMEM((NB, BLK), jnp.int32) for _ in range(4)]
        + [pltpu.VMEM((ROWS, DIM), jnp.float32)]
        + [pltpu.SemaphoreType.DMA, pltpu.SemaphoreType.DMA]
    ),
)
def _sc_lookup(ids_hbm, mods_hbm, t0, t1, t2, t3, out_hbm, *scratch):
    ids_v, mods_v = scratch[0], scratch[1]
    idl = scratch[2:6]
    posl = scratch[6:10]
    pos2d = scratch[10:14]
    rows = scratch[14]
    gsem, ssem = scratch[15], scratch[16]
    tables = (t0, t1, t2, t3)

    wid = lax.axis_index("s") * NC + lax.axis_index("c")
    base = wid * C
    pltpu.sync_copy(ids_hbm.at[pl.ds(base, C)], ids_v)
    pltpu.sync_copy(mods_hbm.at[pl.ds(base, C)], mods_v)

    pending = []

    for h in range(2):
        offs = [jnp.zeros((L,), jnp.int32) for _ in range(4)]
        for i in range(H // L):
            sl = pl.ds(h * H + i * L, L)
            iv = ids_v[sl]
            mv = mods_v[sl]
            pv = jnp.arange(L, dtype=jnp.int32) + (base + h * H + i * L)
            for t in range(4):
                m = mv == t
                dst = plsc.cumsum(m.astype(jnp.int32)) + (offs[t] - 1)
                plsc.store_scatter(idl[t], [dst], iv, mask=m)
                plsc.store_scatter(posl[t], [dst], pv, mask=m)
                offs[t] = offs[t] + plsc.all_reduce_population_count(m)

        cnt = [jnp.max(offs[t]) for t in range(4)]

        for t in range(4):

            @pl.when(cnt[t] > 0)
            def _(t=t):
                for g in range(BLK // L):
                    fill = cnt[t] + jnp.arange(L, dtype=jnp.int32) + g * L
                    src = jnp.maximum(fill - BLK, 0)
                    plsc.store_scatter(idl[t], [fill], plsc.load_gather(idl[t], [src]))
                    plsc.store_scatter(posl[t], [fill], plsc.load_gather(posl[t], [src]))

        nbk = [
            lax.shift_right_logical(cnt[t] + (BLK - 1), BLK_SHIFT) for t in range(4)
        ]
        seg = [None] * 4
        seg[0] = jnp.int32(0)
        for t in range(1, 4):
            seg[t] = seg[t - 1] + lax.shift_left(nbk[t - 1], BLK_SHIFT)

        for cond, dsl, tp, bp in pending:

            @pl.when(cond)
            def _(dsl=dsl, tp=tp, bp=bp):
                pltpu.make_async_copy(
                    rows.at[dsl], out_hbm.at[pos2d[tp].at[bp]], ssem
                ).wait()
        pending = []

        for t in range(4):
            for b in range(NB):
                for g in range(BLK // L):
                    pos2d[t][b, pl.ds(g * L, L)] = posl[t][pl.ds(b * BLK + g * L, L)]

        for t in range(4):
            for b in range(NB):

                @pl.when(b < nbk[t])
                def _(t=t, b=b):
                    pltpu.async_copy(
                        tables[t].at[idl[t].at[pl.ds(b * BLK, BLK)]],
                        rows.at[pl.ds(seg[t] + b * BLK, BLK)],
                        gsem,
                    )
        for t in range(4):
            for b in range(NB):
                cond = b < nbk[t]
                dsl = pl.ds(seg[t] + b * BLK, BLK)

                @pl.when(cond)
                def _(t=t, b=b, dsl=dsl):
                    pltpu.make_async_copy(
                        tables[t].at[idl[t].at[pl.ds(b * BLK, BLK)]],
                        rows.at[dsl],
                        gsem,
                    ).wait()
                    pltpu.async_copy(rows.at[dsl], out_hbm.at[pos2d[t].at[b]], ssem)

                pending.append((cond, dsl, t, b))

    for cond, dsl, tp, bp in pending:

        @pl.when(cond)
        def _(dsl=dsl, tp=tp, bp=bp):
            pltpu.make_async_copy(
                rows.at[dsl], out_hbm.at[pos2d[tp].at[bp]], ssem
            ).wait()


def kernel(input_ids, modality_ids, text_table, image_table, video_table, audio_table):
    ids = input_ids.reshape(-1)
    mods = modality_ids.reshape(-1)
    out = _sc_lookup(ids, mods, text_table, image_table, video_table, audio_table)
    return out.reshape(B, S, DIM)


# speedup vs baseline: 1.6708x; 1.0419x over previous
import functools

import jax
import jax.numpy as jnp
from jax import lax
from jax.experimental import pallas as pl
from jax.experimental.pallas import tpu as pltpu
from jax.experimental.pallas import tpu_sc as plsc

B, S, DIM = 4, 8192, 128
N = B * S

_info = plsc.get_sparse_core_info()
NC, NS, L = _info.num_cores, _info.num_subcores, _info.num_lanes
NW = NC * NS
C = N // NW
H = C // 2
BLK = 64
BLK_SHIFT = 6
NB = H // BLK
CAP = H + BLK
ROWS = H + 4 * BLK

_mesh = plsc.VectorSubcoreMesh(core_axis_name="c", subcore_axis_name="s")


@functools.partial(
    pl.kernel,
    mesh=_mesh,
    compiler_params=pltpu.CompilerParams(needs_layout_passes=False),
    out_type=jax.ShapeDtypeStruct((N, DIM), jnp.float32),
    scratch_types=(
        [pltpu.VMEM((C,), jnp.int32), pltpu.VMEM((C,), jnp.int32)]
        + [pltpu.VMEM((CAP,), jnp.int32) for _ in range(4)]
        + [pltpu.VMEM((CAP,), jnp.int32) for _ in range(4)]
        + [pltpu.VMEM((NB, BLK), jnp.int32) for _ in range(4)]
        + [pltpu.VMEM((ROWS, DIM), jnp.float32)]
        + [pltpu.SemaphoreType.DMA, pltpu.SemaphoreType.DMA]
    ),
)
def _sc_lookup(ids_hbm, mods_hbm, t0, t1, t2, t3, out_hbm, *scratch):
    ids_v, mods_v = scratch[0], scratch[1]
    idl = scratch[2:6]
    posl = scratch[6:10]
    pos2d = scratch[10:14]
    rows = scratch[14]
    gsem, ssem = scratch[15], scratch[16]
    tables = (t0, t1, t2, t3)

    wid = lax.axis_index("s") * NC + lax.axis_index("c")
    base = wid * C
    pltpu.sync_copy(ids_hbm.at[pl.ds(base, C)], ids_v)
    pltpu.sync_copy(mods_hbm.at[pl.ds(base, C)], mods_v)

    pending = []

    for h in range(2):
        offs = [jnp.zeros((L,), jnp.int32) for _ in range(4)]
        for i in range(H // L):
            sl = pl.ds(h * H + i * L, L)
            iv = ids_v[sl]
            mv = mods_v[sl]
            pv = jnp.arange(L, dtype=jnp.int32) + (base + h * H + i * L)
            for t in range(4):
                m = mv == t
                dst = plsc.cumsum(m.astype(jnp.int32)) + (offs[t] - 1)
                plsc.store_scatter(idl[t], [dst], iv, mask=m)
                plsc.store_scatter(posl[t], [dst], pv, mask=m)
                offs[t] = offs[t] + plsc.all_reduce_population_count(m)

        cnt = [jnp.max(offs[t]) for t in range(4)]

        for t in range(4):

            @pl.when(cnt[t] > 0)
            def _(t=t):
                for g in range(BLK // L):
                    fill = cnt[t] + jnp.arange(L, dtype=jnp.int32) + g * L
                    src = jnp.maximum(fill - BLK, 0)
                    plsc.store_scatter(idl[t], [fill], plsc.load_gather(idl[t], [src]))
                    plsc.store_scatter(posl[t], [fill], plsc.load_gather(posl[t], [src]))

        nbk = [
            lax.shift_right_logical(cnt[t] + (BLK - 1), BLK_SHIFT) for t in range(4)
        ]
        seg = [None] * 4
        seg[0] = jnp.int32(0)
        for t in range(1, 4):
            seg[t] = seg[t - 1] + lax.shift_left(nbk[t - 1], BLK_SHIFT)

        for cond, dsl, tp, bp in pending:

            @pl.when(cond)
            def _(dsl=dsl, tp=tp, bp=bp):
                pltpu.make_async_copy(
                    rows.at[dsl], out_hbm.at[pos2d[tp].at[bp]], ssem
                ).wait()
        pending = []

        for t in range(4):
            for b in range(NB):
                for g in range(BLK // L):
                    pos2d[t][b, pl.ds(g * L, L)] = posl[t][pl.ds(b * BLK + g * L, L)]

        for t in range(4):
            for b in range(NB):

                @pl.when(b < nbk[t])
                def _(t=t, b=b):
                    pltpu.async_copy(
                        tables[t].at[idl[t].at[pl.ds(b * BLK, BLK)]],
                        rows.at[pl.ds(seg[t] + b * BLK, BLK)],
                        gsem,
                    )
        for t in range(4):
            for b in range(NB):
                cond = b < nbk[t]
                dsl = pl.ds(seg[t] + b * BLK, BLK)

                @pl.when(cond)
                def _(t=t, b=b, dsl=dsl):
                    pltpu.make_async_copy(
                        tables[t].at[idl[t].at[pl.ds(b * BLK, BLK)]],
                        rows.at[dsl],
                        gsem,
                    ).wait()
                    pltpu.async_copy(rows.at[dsl], out_hbm.at[pos2d[t].at[b]], ssem)

                pending.append((cond, dsl, t, b))

    for cond, dsl, tp, bp in pending:

        @pl.when(cond)
        def _(dsl=dsl, tp=tp, bp=bp):
            pltpu.make_async_copy(
                rows.at[dsl], out_hbm.at[pos2d[tp].at[bp]], ssem
            ).wait()


def kernel(input_ids, modality_ids, text_table, image_table, video_table, audio_table):
    ids = input_ids.reshape(-1)
    mods = modality_ids.reshape(-1)
    out = _sc_lookup(ids, mods, text_table, image_table, video_table, audio_table)
    return out.reshape(B, S, DIM)


# speedup vs baseline: 1.7668x; 1.0574x over previous
import functools

import jax
import jax.numpy as jnp
from jax import lax
from jax.experimental import pallas as pl
from jax.experimental.pallas import tpu as pltpu
from jax.experimental.pallas import tpu_sc as plsc

B, S, DIM = 4, 8192, 128
N = B * S

_info = plsc.get_sparse_core_info()
NC, NS, L = _info.num_cores, _info.num_subcores, _info.num_lanes
NW = NC * NS
C = N // NW
H = C // 2
NB64 = H // 64
NB16 = 4
CAP = H + L
ROWS = H + 4 * L

_mesh = plsc.VectorSubcoreMesh(core_axis_name="c", subcore_axis_name="s")


@functools.partial(
    pl.kernel,
    mesh=_mesh,
    compiler_params=pltpu.CompilerParams(needs_layout_passes=False),
    out_type=jax.ShapeDtypeStruct((N, DIM), jnp.float32),
    scratch_types=(
        [pltpu.VMEM((C,), jnp.int32), pltpu.VMEM((C,), jnp.int32)]
        + [pltpu.VMEM((CAP,), jnp.int32) for _ in range(4)]
        + [pltpu.VMEM((CAP,), jnp.int32) for _ in range(4)]
        + [pltpu.VMEM((NB64, 64), jnp.int32) for _ in range(4)]
        + [pltpu.VMEM((NB16, L), jnp.int32) for _ in range(4)]
        + [pltpu.VMEM((ROWS, DIM), jnp.float32)]
        + [pltpu.SemaphoreType.DMA, pltpu.SemaphoreType.DMA]
    ),
)
def _sc_lookup(ids_hbm, mods_hbm, t0, t1, t2, t3, out_hbm, *scratch):
    ids_v, mods_v = scratch[0], scratch[1]
    idl = scratch[2:6]
    posl = scratch[6:10]
    p2d64 = scratch[10:14]
    p2d16 = scratch[14:18]
    rows = scratch[18]
    gsem, ssem = scratch[19], scratch[20]
    tables = (t0, t1, t2, t3)

    wid = lax.axis_index("s") * NC + lax.axis_index("c")
    base = wid * C
    pltpu.sync_copy(ids_hbm.at[pl.ds(base, C)], ids_v)
    pltpu.sync_copy(mods_hbm.at[pl.ds(base, C)], mods_v)

    pending = []

    for h in range(2):
        offs = [jnp.zeros((L,), jnp.int32) for _ in range(4)]
        for i in range(H // L):
            sl = pl.ds(h * H + i * L, L)
            iv = ids_v[sl]
            mv = mods_v[sl]
            pv = jnp.arange(L, dtype=jnp.int32) + (base + h * H + i * L)
            for t in range(4):
                m = mv == t
                dst = plsc.cumsum(m.astype(jnp.int32)) + (offs[t] - 1)
                plsc.store_scatter(idl[t], [dst], iv, mask=m)
                plsc.store_scatter(posl[t], [dst], pv, mask=m)
                offs[t] = offs[t] + plsc.all_reduce_population_count(m)

        cnt = [jnp.max(offs[t]) for t in range(4)]

        for t in range(4):

            @pl.when(cnt[t] > 0)
            def _(t=t):
                fill = cnt[t] + jnp.arange(L, dtype=jnp.int32)
                src = jnp.maximum(fill - L, 0)
                plsc.store_scatter(idl[t], [fill], plsc.load_gather(idl[t], [src]))
                plsc.store_scatter(posl[t], [fill], plsc.load_gather(posl[t], [src]))

        nb64 = [lax.shift_right_logical(cnt[t], 6) for t in range(4)]
        b64 = [pl.multiple_of(lax.shift_left(nb64[t], 6), 64) for t in range(4)]
        nb16 = [lax.shift_right_logical(cnt[t] - b64[t] + (L - 1), 4) for t in range(4)]
        size = [b64[t] + lax.shift_left(nb16[t], 4) for t in range(4)]
        seg = [None] * 4
        seg[0] = jnp.int32(0)
        for t in range(1, 4):
            seg[t] = pl.multiple_of(seg[t - 1] + size[t - 1], 16)

        for cond, dsl, pref, bp in pending:

            @pl.when(cond)
            def _(dsl=dsl, pref=pref, bp=bp):
                pltpu.make_async_copy(
                    rows.at[dsl], out_hbm.at[pref.at[bp]], ssem
                ).wait()
        pending = []

        for t in range(4):
            for b in range(NB64):
                for g in range(64 // L):
                    p2d64[t][b, pl.ds(g * L, L)] = posl[t][pl.ds(b * 64 + g * L, L)]
            for k in range(NB16):
                off = pl.multiple_of(jnp.minimum(b64[t] + k * L, jnp.int32(H)), 16)
                p2d16[t][k, pl.ds(0, L)] = posl[t][pl.ds(off, L)]

        blocks = []
        for t in range(4):
            for b in range(NB64):
                blocks.append(
                    (b < nb64[t], pl.ds(b * 64, 64),
                     pl.ds(pl.multiple_of(seg[t] + b * 64, 16), 64), p2d64[t], b, t)
                )
            for k in range(NB16):
                blocks.append(
                    (k < nb16[t], pl.ds(pl.multiple_of(b64[t] + k * L, 16), L),
                     pl.ds(pl.multiple_of(seg[t] + b64[t] + k * L, 16), L), p2d16[t], k, t)
                )

        for cond, isl, dsl, pref, bp, t in blocks:

            @pl.when(cond)
            def _(isl=isl, dsl=dsl, t=t):
                pltpu.async_copy(tables[t].at[idl[t].at[isl]], rows.at[dsl], gsem)
        for cond, isl, dsl, pref, bp, t in blocks:

            @pl.when(cond)
            def _(isl=isl, dsl=dsl, pref=pref, bp=bp, t=t):
                pltpu.make_async_copy(
                    tables[t].at[idl[t].at[isl]], rows.at[dsl], gsem
                ).wait()
                pltpu.async_copy(rows.at[dsl], out_hbm.at[pref.at[bp]], ssem)

            pending.append((cond, dsl, pref, bp))

    for cond, dsl, pref, bp in pending:

        @pl.when(cond)
        def _(dsl=dsl, pref=pref, bp=bp):
            pltpu.make_async_copy(
                rows.at[dsl], out_hbm.at[pref.at[bp]], ssem
            ).wait()


def kernel(input_ids, modality_ids, text_table, image_table, video_table, audio_table):
    ids = input_ids.reshape(-1)
    mods = modality_ids.reshape(-1)
    out = _sc_lookup(ids, mods, text_table, image_table, video_table, audio_table)
    return out.reshape(B, S, DIM)
